# Initial kernel scaffold; baseline (speedup 1.0000x reference)
import jax
import jax.numpy as jnp
from jax.experimental import pallas as pl


def kernel(x, W):
    raise NotImplementedError("write your pallas kernel here")


# speedup vs baseline: 1.0065x; 1.0065x over previous
import functools

import jax
import jax.numpy as jnp
from jax import lax
from jax.experimental import pallas as pl
from jax.experimental.pallas import tpu as pltpu
from jax.experimental.pallas import tpu_sc as plsc

_N = 8192
_D = 4096
_TH = 0.5
_BC = 512
_NB = _N // _BC
_L = 16


def _tc_body(x_ref, w_ref, mask_ref, logits_ref):
    i = pl.program_id(0)
    blk = jnp.dot(x_ref[...], w_ref[...], preferred_element_type=jnp.float32)
    logits_ref[:, pl.ds(i * _BC, _BC)] = blk

    @pl.when(i == _NB - 1)
    def _():
        lg = logits_ref[...]
        m = jnp.max(lg)
        e = jnp.exp(lg - m)
        s = jnp.sum(e)
        scores = e / s
        mask_ref[...] = (scores > (_TH / _N)).astype(jnp.int32)


_tc_mask = pl.pallas_call(
    _tc_body,
    grid=(_NB,),
    in_specs=[
        pl.BlockSpec((1, _D), lambda i: (0, 0)),
        pl.BlockSpec((_D, _BC), lambda i: (0, i)),
    ],
    out_specs=pl.BlockSpec((1, _N), lambda i: (0, 0)),
    out_shape=jax.ShapeDtypeStruct((1, _N), jnp.int32),
    scratch_shapes=[pltpu.VMEM((1, _N), jnp.float32)],
)


def _lane_gather(v, src):
    return lax.gather(
        v,
        src[:, None],
        lax.GatherDimensionNumbers(
            offset_dims=(), collapsed_slice_dims=(0,), start_index_map=(0,)
        ),
        slice_sizes=(1,),
        mode=lax.GatherScatterMode.PROMISE_IN_BOUNDS,
    )


def _lane_shift_down(v, k, lanes):
    shifted = _lane_gather(v, jnp.maximum(lanes - k, 0))
    return jnp.where(lanes >= k, shifted, 0)


def _sc_body(mask_hbm, out_hbm, mask_v, out_v):
    cid = lax.axis_index("c")
    sid = lax.axis_index("s")

    @pl.when((cid == 0) & (sid == 0))
    def _():
        pltpu.sync_copy(mask_hbm, mask_v)
        lanes = lax.iota(jnp.int32, _L)

        def fill(i, c):
            out_v[pl.ds(i * _L, _L)] = jnp.full((_L,), -1, jnp.int32)
            return c

        lax.fori_loop(0, _N // _L, fill, 0)

        def body(i, off):
            m = mask_v[pl.ds(i * _L, _L)]
            mb = m > 0
            incl = m
            for k in (1, 2, 4, 8):
                incl = incl + _lane_shift_down(incl, k, lanes)
            idx = lanes + i * _L
            pos = (off + incl) - m
            plsc.store_scatter(out_v, [pos], idx, mask=mb)
            total = _lane_gather(incl, lanes * 0 + (_L - 1))
            return off + total

        lax.fori_loop(0, _N // _L, body, jnp.zeros((_L,), jnp.int32))
        pltpu.sync_copy(out_v, out_hbm)


@functools.cache
def _sc_compact():
    return pl.kernel(
        _sc_body,
        mesh=plsc.VectorSubcoreMesh(core_axis_name="c", subcore_axis_name="s"),
        out_type=jax.ShapeDtypeStruct((_N,), jnp.int32),
        scratch_types=[
            pltpu.VMEM((_N,), jnp.int32),
            pltpu.VMEM((_N,), jnp.int32),
        ],
        compiler_params=pltpu.CompilerParams(needs_layout_passes=False),
    )


def kernel(x, W):
    mask2d = _tc_mask(x.reshape(1, _D), W)
    inds = _sc_compact()(mask2d.reshape(_N))
    return inds, mask2d
